# baseline (speedup 1.0000x reference)
import jax
import jax.numpy as jnp
from jax.experimental import pallas as pl
from jax.experimental.pallas import tpu as pltpu

_N_ROWS = 4096
_N_COLS = 4096
_BLOCK_ROWS = 256
_GRID = _N_ROWS // _BLOCK_ROWS
_TOTAL = _N_ROWS * _N_COLS


def _focal_body(x_ref, t_ref, out_ref):
    i = pl.program_id(0)
    x = x_ref[...]
    t = t_ref[...]
    p = jnp.where(t != 0, x, 1.0 - x)
    one_m = 1.0 - p
    term = one_m * one_m * jnp.log(p)
    s = -jnp.sum(term)

    @pl.when(i == 0)
    def _init():
        out_ref[0, 0] = s

    @pl.when(i != 0)
    def _acc():
        out_ref[0, 0] += s


def kernel(inputs, targets):
    partial = pl.pallas_call(
        _focal_body,
        grid=(_GRID,),
        in_specs=[
            pl.BlockSpec((_BLOCK_ROWS, _N_COLS), lambda i: (i, 0)),
            pl.BlockSpec((_BLOCK_ROWS, _N_COLS), lambda i: (i, 0)),
        ],
        out_specs=pl.BlockSpec(memory_space=pltpu.SMEM),
        out_shape=jax.ShapeDtypeStruct((1, 1), jnp.float32),
    )(inputs, targets)
    return partial[0, 0] / _TOTAL
